# baseline (speedup 1.0000x reference)
import jax
import jax.numpy as jnp
from jax.experimental import pallas as pl
from jax.experimental.pallas import tpu as pltpu

_TILE = 512


def _fused_body(x_ref, wr_ref, br_ref, we_hbm, be_ref, out_hbm,
                g_ref, sg_ref, obuf_ref, we_ref, we_sem, out_sem,
                *, n_chunks, n_tokens):
    i = pl.program_id(0)

    @pl.when(i == 0)
    def _start_we_copy():
        pltpu.make_async_copy(we_hbm, we_ref, we_sem).start()

    @pl.when(i < n_chunks)
    def _phase1():
        x = x_ref[...]
        lt = jax.lax.dot_general(
            wr_ref[...], x, (((1,), (1,)), ((), ())),
            preferred_element_type=jnp.float32,
            precision=jax.lax.Precision.DEFAULT) + br_ref[...]
        m = jnp.max(lt, axis=0, keepdims=True)
        gmax = 1.0 / jnp.sum(jnp.exp(lt - m), axis=0, keepdims=True)
        gpart = jax.lax.dot_general(
            gmax, x, (((1,), (0,)), ((), ())),
            preferred_element_type=jnp.float32,
            precision=jax.lax.Precision.DEFAULT)
        sgpart = jnp.sum(gmax)

        @pl.when(i == 0)
        def _():
            g_ref[...] = gpart
            sg_ref[0, 0] = sgpart

        @pl.when(i > 0)
        def _():
            g_ref[...] = g_ref[...] + gpart
            sg_ref[0, 0] = sg_ref[0, 0] + sgpart

    @pl.when(i == n_chunks)
    def _epilogue():
        pltpu.make_async_copy(we_hbm, we_ref, we_sem).wait()
        v = jax.lax.dot_general(
            g_ref[...], we_ref[...], (((1,), (1,)), ((), ())),
            preferred_element_type=jnp.float32) + sg_ref[0, 0] * be_ref[...]
        obuf_ref[...] = jnp.broadcast_to(v, obuf_ref.shape)
        for k in range(n_tokens // _TILE):
            pltpu.make_async_copy(
                obuf_ref, out_hbm.at[pl.ds(k * _TILE, _TILE), :],
                out_sem).start()
        for k in range(n_tokens // _TILE):
            pltpu.make_async_copy(
                obuf_ref, out_hbm.at[pl.ds(k * _TILE, _TILE), :],
                out_sem).wait()


def kernel(x, W_r, b_r, W_e, b_e):
    batch, seq, hidden = x.shape
    n_tokens = batch * seq
    xf = x.reshape(n_tokens, hidden)
    n_chunks = n_tokens // _TILE

    yf = pl.pallas_call(
        lambda *refs: _fused_body(*refs, n_chunks=n_chunks, n_tokens=n_tokens),
        grid=(n_chunks + 1,),
        in_specs=[
            pl.BlockSpec((_TILE, hidden),
                         lambda i: (jnp.minimum(i, n_chunks - 1), 0)),
            pl.BlockSpec((W_r.shape[0], hidden), lambda i: (0, 0)),
            pl.BlockSpec((W_r.shape[0], 1), lambda i: (0, 0)),
            pl.BlockSpec(memory_space=pl.ANY),
            pl.BlockSpec((1, hidden), lambda i: (0, 0)),
        ],
        out_specs=pl.BlockSpec(memory_space=pl.ANY),
        out_shape=jax.ShapeDtypeStruct((n_tokens, hidden), jnp.float32),
        scratch_shapes=[
            pltpu.VMEM((1, hidden), jnp.float32),
            pltpu.SMEM((1, 1), jnp.float32),
            pltpu.VMEM((_TILE, hidden), jnp.float32),
            pltpu.VMEM((hidden, hidden), jnp.float32),
            pltpu.SemaphoreType.DMA,
            pltpu.SemaphoreType.DMA,
        ],
    )(xf, W_r, b_r.reshape(-1, 1), W_e, b_e.reshape(1, -1))

    return yf.reshape(batch, seq, hidden)
